# baseline (speedup 1.0000x reference)
import functools

import jax
import jax.numpy as jnp
from jax import lax
from jax.experimental import pallas as pl
from jax.experimental.pallas import tpu as pltpu
from jax.experimental.pallas import tpu_sc as plsc

_N = 10000
_E = 320000
_D = 128
_R = 200
_RH = _R // 2

_NC = 2
_NS = 16
_DH = _D // _NC
_CHUNK = 128
_CPT = 160
_NCHUNK = _NS * _CPT
_EPAD = _NCHUNK * _CHUNK
_NPAD = 10240
_RPT = _NPAD // _NS
_HW = _NPAD * _RH
_HTRASH = (_NPAD - 1) * _RH
_HB = 8000
_TRASH = _NPAD - 1


def _tables_body(x_ref, w_ref, o_ref):
    o_ref[...] = jnp.dot(x_ref[...], w_ref[0], preferred_element_type=jnp.float32)


def _build_T(n_feats, w_stack):
    return pl.pallas_call(
        _tables_body,
        grid=(2, 10),
        in_specs=[
            pl.BlockSpec((_N // 10, _D), lambda i, j: (j, 0)),
            pl.BlockSpec((1, _D, _D), lambda i, j: (i, 0, 0)),
        ],
        out_specs=pl.BlockSpec((_N // 10, _D), lambda i, j: (i * 10 + j, 0)),
        out_shape=jax.ShapeDtypeStruct((2 * _N, _D), jnp.float32),
    )(n_feats, w_stack)


def _rel_body(r_ref, wI_ref, wO_ref, wR_ref, bI_ref, bO_ref, bR_ref,
              qneg_ref, rout_ref):
    r = r_ref[...]
    rI = jnp.dot(r, wI_ref[...], preferred_element_type=jnp.float32)
    rO = jnp.dot(r, wO_ref[...], preferred_element_type=jnp.float32)
    rR = jnp.dot(r, wR_ref[...], preferred_element_type=jnp.float32)
    rowid = lax.broadcasted_iota(jnp.int32, (_R, _D), 0)
    qneg_ref[...] = jnp.where(rowid < _RH, bI_ref[...] - rI, bO_ref[...] - rO)
    rout_ref[...] = rR + bR_ref[...]


def _build_rel(r_feats, wI, wO, wR, bI, bO, bR):
    return pl.pallas_call(
        _rel_body,
        out_shape=(
            jax.ShapeDtypeStruct((_R, _D), jnp.float32),
            jax.ShapeDtypeStruct((_R, _D), jnp.float32),
        ),
    )(r_feats, wI, wO, wR, bI.reshape(1, _D), bO.reshape(1, _D),
      bR.reshape(1, _D))


def _gidx_body(src_ref, et_ref, dst_ref, g_ref, h0_ref, h1_ref):
    et = et_ref[...]
    dst = dst_ref[...]
    g_ref[...] = src_ref[...] + jnp.where(et >= _RH, _N, 0)
    base = dst * _RH
    h0_ref[...] = jnp.where(et < _RH, base + et, _HTRASH)
    h1_ref[...] = jnp.where(et >= _RH, base + (et - _RH), _HTRASH)


def _build_idx(src2, et2, dst2):
    return pl.pallas_call(
        _gidx_body,
        out_shape=(
            jax.ShapeDtypeStruct(src2.shape, jnp.int32),
            jax.ShapeDtypeStruct(src2.shape, jnp.int32),
            jax.ShapeDtypeStruct(src2.shape, jnp.int32),
        ),
    )(src2, et2, dst2)


def _make_sc_kernel():
    mesh = plsc.VectorSubcoreMesh(core_axis_name="c", subcore_axis_name="s")

    @functools.partial(
        pl.kernel,
        out_type=(
            jax.ShapeDtypeStruct((_NC, _NPAD, _DH), jnp.float32),
            jax.ShapeDtypeStruct((_NC, _HW), jnp.float32),
        ),
        mesh=mesh,
        compiler_params=pltpu.CompilerParams(use_tc_tiling_on_sc=False),
        scratch_types=[
            pltpu.VMEM((_CHUNK,), jnp.int32),
            pltpu.VMEM((_CHUNK,), jnp.int32),
            pltpu.VMEM((_CHUNK,), jnp.int32),
            pltpu.VMEM((_CHUNK,), jnp.int32),
            pltpu.VMEM((_CHUNK,), jnp.int32),
            pltpu.VMEM((_CHUNK,), jnp.int32),
            pltpu.VMEM((_CHUNK, _DH), jnp.float32),
            pltpu.VMEM((_CHUNK, _DH), jnp.float32),
            pltpu.VMEM((_HB,), jnp.float32),
            pltpu.VMEM((_CHUNK,), jnp.float32),
            pltpu.VMEM_SHARED((_NPAD, _DH), jnp.float32),
            pltpu.VMEM_SHARED((_HW,), jnp.float32),
            pltpu.SemaphoreType.DMA,
            pltpu.SemaphoreType.DMA,
            pltpu.SemaphoreType.DMA,
            pltpu.SemaphoreType.DMA,
        ],
    )
    def sc_kernel(T_hbm, gidx_hbm, dst_hbm, hidx_hbm,
                  acc_hbm, hist_hbm,
                  gidx_a, dst_a, hidx_a, gidx_b, dst_b, hidx_b,
                  rows_a, rows_b, zb_v, ones_v,
                  acc_s, hist_s, semia, semib, semga, semgb):
        cid = lax.axis_index("c")
        sid = lax.axis_index("s")
        r0 = sid * _RPT
        h0 = sid * (_HW // _NS)

        zero16 = jnp.zeros((16,), jnp.float32)
        one16 = jnp.ones((16,), jnp.float32)

        def _fillz(i, carry):
            zb_v[pl.ds(i * 16, 16)] = zero16
            return carry

        lax.fori_loop(0, _HB // 16, _fillz, 0)

        def _fillr(i, carry):
            for c8 in range(_DH // 16):
                rows_a[i, pl.ds(c8 * 16, 16)] = zero16
            return carry

        lax.fori_loop(0, _CHUNK, _fillr, 0)
        for c8 in range(_CHUNK // 16):
            ones_v[pl.ds(c8 * 16, 16)] = one16

        for b in range(_RPT // _CHUNK):
            off = r0 + b * _CHUNK
            pltpu.sync_copy(rows_a, acc_s.at[pl.ds(off, _CHUNK)])
        for b in range(_HW // _NS // _HB):
            pltpu.sync_copy(zb_v, hist_s.at[pl.ds(h0 + b * _HB, _HB)])
        plsc.subcore_barrier()

        lo = sid * _CPT
        hi = lo + _CPT

        def _issue_idx(c, g_v, d_v, h_v, sem):
            base = pl.multiple_of(c * _CHUNK, _CHUNK)
            pltpu.async_copy(gidx_hbm.at[pl.ds(base, _CHUNK)], g_v, sem)
            pltpu.async_copy(dst_hbm.at[pl.ds(base, _CHUNK)], d_v, sem)
            pltpu.async_copy(hidx_hbm.at[cid, pl.ds(base, _CHUNK)], h_v, sem)

        def _wait_idx(g_v, d_v, h_v, sem):
            pltpu.make_async_copy(gidx_hbm.at[pl.ds(0, _CHUNK)], g_v,
                                  sem).wait()
            pltpu.make_async_copy(dst_hbm.at[pl.ds(0, _CHUNK)], d_v,
                                  sem).wait()
            pltpu.make_async_copy(hidx_hbm.at[cid, pl.ds(0, _CHUNK)], h_v,
                                  sem).wait()

        _issue_idx(lo, gidx_a, dst_a, hidx_a, semia)
        _issue_idx(lo + 1, gidx_b, dst_b, hidx_b, semib)

        def _pair(k, carry):
            c0 = lo + 2 * k
            _wait_idx(gidx_a, dst_a, hidx_a, semia)
            ga = pltpu.async_copy(T_hbm.at[cid].at[gidx_a], rows_a, semga)
            _wait_idx(gidx_b, dst_b, hidx_b, semib)
            gb = pltpu.async_copy(T_hbm.at[cid].at[gidx_b], rows_b, semgb)
            ga.wait()
            pltpu.sync_copy(rows_a, acc_s.at[dst_a], add=True)
            pltpu.sync_copy(ones_v, hist_s.at[hidx_a], add=True)
            ca = lax.min(c0 + 2, hi - 1)
            _issue_idx(ca, gidx_a, dst_a, hidx_a, semia)
            gb.wait()
            pltpu.sync_copy(rows_b, acc_s.at[dst_b], add=True)
            pltpu.sync_copy(ones_v, hist_s.at[hidx_b], add=True)
            cb = lax.min(c0 + 3, hi - 1)
            _issue_idx(cb, gidx_b, dst_b, hidx_b, semib)
            return carry

        lax.fori_loop(0, _CPT // 2, _pair, 0)
        _wait_idx(gidx_a, dst_a, hidx_a, semia)
        _wait_idx(gidx_b, dst_b, hidx_b, semib)
        plsc.subcore_barrier()

        for b in range(_RPT // _CHUNK):
            off = r0 + b * _CHUNK
            pltpu.sync_copy(acc_s.at[pl.ds(off, _CHUNK)], rows_a)
            pltpu.sync_copy(rows_a, acc_hbm.at[cid, pl.ds(off, _CHUNK)])
        for b in range(_HW // _NS // _HB):
            off = h0 + b * _HB
            pltpu.sync_copy(hist_s.at[pl.ds(off, _HB)], zb_v)
            pltpu.sync_copy(zb_v, hist_hbm.at[cid, pl.ds(off, _HB)])

    return sc_kernel


def _mean_body(acc_ref, h0_ref, h1_ref, q0_ref, q1_ref, o_ref):
    h0 = h0_ref[...]
    h1 = h1_ref[...]
    q = (jnp.dot(h0, q0_ref[...], preferred_element_type=jnp.float32)
         + jnp.dot(h1, q1_ref[...], preferred_element_type=jnp.float32))
    cnt = jnp.sum(h0 + h1, axis=1, keepdims=True)
    o_ref[...] = (acc_ref[...] + q) / jnp.maximum(cnt, 1.0)


def _segment_mean(acc_full, h0, h1, q0, q1):
    nb = 5
    rb = _N // nb
    return pl.pallas_call(
        _mean_body,
        grid=(nb,),
        in_specs=[
            pl.BlockSpec((rb, _D), lambda i: (i, 0)),
            pl.BlockSpec((rb, _D), lambda i: (i, 0)),
            pl.BlockSpec((rb, _D), lambda i: (i, 0)),
            pl.BlockSpec((_D, _D), lambda i: (0, 0)),
            pl.BlockSpec((_D, _D), lambda i: (0, 0)),
        ],
        out_specs=pl.BlockSpec((rb, _D), lambda i: (i, 0)),
        out_shape=jax.ShapeDtypeStruct((_N, _D), jnp.float32),
    )(acc_full, h0, h1, q0, q1)


def kernel(n_feats, edge_index, etype, r_feats, num_rels,
           W_I_w, W_I_b, W_O_w, W_O_b, W_R_w, W_R_b):
    w_stack = jnp.stack([W_I_w, W_O_w])
    T = _build_T(n_feats, w_stack)
    qneg, r_out = _build_rel(r_feats, W_I_w, W_O_w, W_R_w, W_I_b, W_O_b, W_R_b)

    T_split = jnp.stack([T[:, :_DH], T[:, _DH:]])
    q_pad = jnp.zeros((_NC, _D, _D), jnp.float32).at[:, :_RH, :].set(
        qneg.reshape(_NC, _RH, _D))

    npadE = _EPAD - _E
    src_p = jnp.concatenate([edge_index[0], jnp.zeros((npadE,), jnp.int32)])
    et_p = jnp.concatenate([etype, jnp.zeros((npadE,), jnp.int32)])
    dst_p = jnp.concatenate(
        [edge_index[1], jnp.full((npadE,), _TRASH, jnp.int32)])

    src2 = src_p.reshape(_NCHUNK, _CHUNK)
    et2 = et_p.reshape(_NCHUNK, _CHUNK)
    dst2 = dst_p.reshape(_NCHUNK, _CHUNK)
    gidx, hidx0, hidx1 = _build_idx(src2, et2, dst2)
    gidx = gidx.reshape(_EPAD)
    hidx = jnp.stack([hidx0.reshape(_EPAD), hidx1.reshape(_EPAD)])

    sc = _make_sc_kernel()
    acc, hist = sc(T_split, gidx, dst_p, hidx)

    acc_full = jnp.concatenate([acc[0], acc[1]], axis=1)
    hist = hist.reshape(_NC, _NPAD, _RH)
    h_pad = jnp.zeros((_NC, _NPAD, _D), jnp.float32).at[:, :, :_RH].set(hist)

    n_out = _segment_mean(acc_full, h_pad[0], h_pad[1], q_pad[0], q_pad[1])
    return (n_out, r_out)


# speedup vs baseline: 1.1074x; 1.1074x over previous
import functools

import jax
import jax.numpy as jnp
from jax import lax
from jax.experimental import pallas as pl
from jax.experimental.pallas import tpu as pltpu
from jax.experimental.pallas import tpu_sc as plsc

_N = 10000
_E = 320000
_D = 128
_R = 200
_RH = _R // 2

_NC = 2
_NS = 16
_DH = _D // _NC
_CHUNK = 128
_NCHUNK = _E // _CHUNK
_NPAD = 10240
_RPT = _NPAD // _NS
_HW = _NPAD * _RH
_HTRASH = (_NPAD - 1) * _RH
_HB = 8000


def _tables_body(x_ref, w_ref, o_ref):
    o_ref[...] = jnp.dot(x_ref[...], w_ref[0], preferred_element_type=jnp.float32)


def _build_T(n_feats, w_stack):
    return pl.pallas_call(
        _tables_body,
        grid=(2, 10),
        in_specs=[
            pl.BlockSpec((_N // 10, _D), lambda i, j: (j, 0)),
            pl.BlockSpec((1, _D, _D), lambda i, j: (i, 0, 0)),
        ],
        out_specs=pl.BlockSpec((_N // 10, _D), lambda i, j: (i * 10 + j, 0)),
        out_shape=jax.ShapeDtypeStruct((2 * _N, _D), jnp.float32),
    )(n_feats, w_stack)


def _rel_body(r_ref, wI_ref, wO_ref, wR_ref, bI_ref, bO_ref, bR_ref,
              qneg_ref, rout_ref):
    r = r_ref[...]
    rI = jnp.dot(r, wI_ref[...], preferred_element_type=jnp.float32)
    rO = jnp.dot(r, wO_ref[...], preferred_element_type=jnp.float32)
    rR = jnp.dot(r, wR_ref[...], preferred_element_type=jnp.float32)
    rowid = lax.broadcasted_iota(jnp.int32, (_R, _D), 0)
    qneg_ref[...] = jnp.where(rowid < _RH, bI_ref[...] - rI, bO_ref[...] - rO)
    rout_ref[...] = rR + bR_ref[...]


def _build_rel(r_feats, wI, wO, wR, bI, bO, bR):
    return pl.pallas_call(
        _rel_body,
        out_shape=(
            jax.ShapeDtypeStruct((_R, _D), jnp.float32),
            jax.ShapeDtypeStruct((_R, _D), jnp.float32),
        ),
    )(r_feats, wI, wO, wR, bI.reshape(1, _D), bO.reshape(1, _D),
      bR.reshape(1, _D))


def _gidx_body(src_ref, et_ref, dst_ref, g_ref, h0_ref, h1_ref):
    et = et_ref[...]
    dst = dst_ref[...]
    g_ref[...] = src_ref[...] + jnp.where(et >= _RH, _N, 0)
    base = dst * _RH
    h0_ref[...] = jnp.where(et < _RH, base + et, _HTRASH)
    h1_ref[...] = jnp.where(et >= _RH, base + (et - _RH), _HTRASH)


def _build_idx(src2, et2, dst2):
    return pl.pallas_call(
        _gidx_body,
        out_shape=(
            jax.ShapeDtypeStruct(src2.shape, jnp.int32),
            jax.ShapeDtypeStruct(src2.shape, jnp.int32),
            jax.ShapeDtypeStruct(src2.shape, jnp.int32),
        ),
    )(src2, et2, dst2)


def _make_sc_kernel():
    mesh = plsc.VectorSubcoreMesh(core_axis_name="c", subcore_axis_name="s")

    @functools.partial(
        pl.kernel,
        out_type=(
            jax.ShapeDtypeStruct((_NC, _NPAD, _DH), jnp.float32),
            jax.ShapeDtypeStruct((_NC, _HW), jnp.float32),
        ),
        mesh=mesh,
        compiler_params=pltpu.CompilerParams(use_tc_tiling_on_sc=False),
        scratch_types=[
            pltpu.VMEM((_CHUNK,), jnp.int32),
            pltpu.VMEM((_CHUNK,), jnp.int32),
            pltpu.VMEM((_CHUNK,), jnp.int32),
            pltpu.VMEM((_CHUNK, _DH), jnp.float32),
            pltpu.VMEM((_HB,), jnp.float32),
            pltpu.VMEM((_CHUNK,), jnp.float32),
            pltpu.VMEM_SHARED((_NPAD, _DH), jnp.float32),
            pltpu.VMEM_SHARED((_HW,), jnp.float32),
            pltpu.SemaphoreType.DMA,
            pltpu.SemaphoreType.DMA,
        ],
    )
    def sc_kernel(T_hbm, gidx_hbm, dst_hbm, hidx_hbm,
                  acc_hbm, hist_hbm,
                  gidx_v, dst_v, hidx_v, rows_v, zb_v, ones_v,
                  acc_s, hist_s, semi, semg):
        cid = lax.axis_index("c")
        sid = lax.axis_index("s")
        r0 = sid * _RPT
        h0 = sid * (_HW // _NS)

        zero16 = jnp.zeros((16,), jnp.float32)
        one16 = jnp.ones((16,), jnp.float32)

        def _fillz(i, carry):
            zb_v[pl.ds(i * 16, 16)] = zero16
            return carry

        lax.fori_loop(0, _HB // 16, _fillz, 0)

        def _fillr(i, carry):
            for c8 in range(_DH // 16):
                rows_v[i, pl.ds(c8 * 16, 16)] = zero16
            return carry

        lax.fori_loop(0, _CHUNK, _fillr, 0)
        for c8 in range(_CHUNK // 16):
            ones_v[pl.ds(c8 * 16, 16)] = one16

        for b in range(_RPT // _CHUNK):
            off = r0 + b * _CHUNK
            pltpu.sync_copy(rows_v, acc_s.at[pl.ds(off, _CHUNK)])
        for b in range(_HW // _NS // _HB):
            pltpu.sync_copy(zb_v, hist_s.at[pl.ds(h0 + b * _HB, _HB)])
        plsc.subcore_barrier()

        lo = sid * _NCHUNK // _NS
        hi = (sid + 1) * _NCHUNK // _NS

        def _chunk(c, carry):
            base = pl.multiple_of(c * _CHUNK, _CHUNK)
            ci = pltpu.async_copy(gidx_hbm.at[pl.ds(base, _CHUNK)], gidx_v,
                                  semi)
            cd = pltpu.async_copy(dst_hbm.at[pl.ds(base, _CHUNK)], dst_v, semi)
            ch = pltpu.async_copy(hidx_hbm.at[cid, pl.ds(base, _CHUNK)],
                                  hidx_v, semi)
            ci.wait()
            cd.wait()
            ch.wait()
            g1 = pltpu.async_copy(T_hbm.at[cid].at[gidx_v], rows_v, semg)
            g1.wait()
            pltpu.sync_copy(rows_v, acc_s.at[dst_v], add=True)
            pltpu.sync_copy(ones_v, hist_s.at[hidx_v], add=True)
            return carry

        lax.fori_loop(lo, hi, _chunk, 0)
        plsc.subcore_barrier()

        for b in range(_RPT // _CHUNK):
            off = r0 + b * _CHUNK
            pltpu.sync_copy(acc_s.at[pl.ds(off, _CHUNK)], rows_v)
            pltpu.sync_copy(rows_v, acc_hbm.at[cid, pl.ds(off, _CHUNK)])
        for b in range(_HW // _NS // _HB):
            off = h0 + b * _HB
            pltpu.sync_copy(hist_s.at[pl.ds(off, _HB)], zb_v)
            pltpu.sync_copy(zb_v, hist_hbm.at[cid, pl.ds(off, _HB)])

    return sc_kernel


def _mean_body(acc_ref, h0_ref, h1_ref, q0_ref, q1_ref, o_ref):
    h0 = h0_ref[...]
    h1 = h1_ref[...]
    q = (jnp.dot(h0, q0_ref[...], preferred_element_type=jnp.float32)
         + jnp.dot(h1, q1_ref[...], preferred_element_type=jnp.float32))
    cnt = jnp.sum(h0 + h1, axis=1, keepdims=True)
    o_ref[...] = (acc_ref[...] + q) / jnp.maximum(cnt, 1.0)


def _segment_mean(acc_full, h0, h1, q0, q1):
    nb = 5
    rb = _NPAD // nb
    return pl.pallas_call(
        _mean_body,
        grid=(nb,),
        in_specs=[
            pl.BlockSpec((rb, _D), lambda i: (i, 0)),
            pl.BlockSpec((rb, _D), lambda i: (i, 0)),
            pl.BlockSpec((rb, _D), lambda i: (i, 0)),
            pl.BlockSpec((_D, _D), lambda i: (0, 0)),
            pl.BlockSpec((_D, _D), lambda i: (0, 0)),
        ],
        out_specs=pl.BlockSpec((rb, _D), lambda i: (i, 0)),
        out_shape=jax.ShapeDtypeStruct((_NPAD, _D), jnp.float32),
    )(acc_full, h0, h1, q0, q1)


def kernel(n_feats, edge_index, etype, r_feats, num_rels,
           W_I_w, W_I_b, W_O_w, W_O_b, W_R_w, W_R_b):
    w_stack = jnp.stack([W_I_w, W_O_w])
    T = _build_T(n_feats, w_stack)
    qneg, r_out = _build_rel(r_feats, W_I_w, W_O_w, W_R_w, W_I_b, W_O_b, W_R_b)

    T_split = jnp.stack([T[:, :_DH], T[:, _DH:]])
    q_pad = jnp.zeros((_NC, _D, _D), jnp.float32).at[:, :_RH, :].set(
        qneg.reshape(_NC, _RH, _D))

    src2 = edge_index[0].reshape(_E // _D, _D)
    et2 = etype.reshape(_E // _D, _D)
    dst2 = edge_index[1].reshape(_E // _D, _D)
    gidx, hidx0, hidx1 = _build_idx(src2, et2, dst2)
    gidx = gidx.reshape(_E)
    hidx = jnp.stack([hidx0.reshape(_E), hidx1.reshape(_E)])

    sc = _make_sc_kernel()
    acc, hist = sc(T_split, gidx, edge_index[1], hidx)

    acc_full = jnp.concatenate([acc[0], acc[1]], axis=1)
    hist = hist.reshape(_NC, _NPAD, _RH)
    h_pad = jnp.zeros((_NC, _NPAD, _D), jnp.float32).at[:, :, :_RH].set(hist)

    n_out = _segment_mean(acc_full, h_pad[0], h_pad[1],
                          q_pad[0], q_pad[1])[:_N]
    return (n_out, r_out)
